# baseline (speedup 1.0000x reference)
import functools

import jax
import jax.numpy as jnp
from jax import lax
from jax.experimental import pallas as pl
from jax.experimental.pallas import tpu as pltpu
from jax.experimental.pallas import tpu_sc as plsc

_B = 16384
_D = 64
_NNEG = 5

_info = plsc.get_sparse_core_info()
_NC = _info.num_cores
_NS = _info.num_subcores
_NW = _NC * _NS

_PC_PER_W = _B // _NW
_NEG_PER_W = _B * _NNEG // _NW
_CHUNK = 512
_N_CHUNKS = (_PC_PER_W + _PC_PER_W + _NEG_PER_W) // _CHUNK


def _sc_body(center_hbm, context_hbm, pc_idx_hbm, pctx_idx_hbm, neg_idx_hbm,
             pc_out, pctx_out, neg_out,
             idx_v, rows_a, rows_b, sem_a, sem_b):
    wid = lax.axis_index("s") * _NC + lax.axis_index("c")
    base = wid * _PC_PER_W
    nbase = wid * _NEG_PER_W

    pltpu.sync_copy(pc_idx_hbm.at[pl.ds(base, _PC_PER_W)],
                    idx_v.at[pl.ds(0, _PC_PER_W)])
    pltpu.sync_copy(pctx_idx_hbm.at[pl.ds(base, _PC_PER_W)],
                    idx_v.at[pl.ds(_PC_PER_W, _PC_PER_W)])
    pltpu.sync_copy(neg_idx_hbm.at[pl.ds(nbase, _NEG_PER_W)],
                    idx_v.at[pl.ds(2 * _PC_PER_W, _NEG_PER_W)])

    plan = [(center_hbm, 0, pc_out, base),
            (context_hbm, _CHUNK, pctx_out, base)]
    for j in range(_NNEG):
        plan.append((context_hbm, (2 + j) * _CHUNK, neg_out, nbase + j * _CHUNK))

    bufs = (rows_a, rows_b)
    sems = (sem_a, sem_b)

    def gather(c):
        table, ioff, _, _ = plan[c]
        return pltpu.async_copy(
            table.at[idx_v.at[pl.ds(ioff, _CHUNK)]], bufs[c % 2], sems[c % 2])

    def store(c):
        _, _, out, ooff = plan[c]
        pltpu.sync_copy(bufs[c % 2], out.at[pl.ds(ooff, _CHUNK)])

    h = gather(0)
    for c in range(1, _N_CHUNKS):
        h_next = gather(c)
        h.wait()
        store(c - 1)
        h = h_next
    h.wait()
    store(_N_CHUNKS - 1)


@jax.jit
def _fasttext_gather(center_W, context_W, pc_idx, pctx_idx, neg_idx):
    mesh = plsc.VectorSubcoreMesh(core_axis_name="c", subcore_axis_name="s")
    return pl.kernel(
        _sc_body,
        mesh=mesh,
        compiler_params=pltpu.CompilerParams(use_tc_tiling_on_sc=False),
        out_type=(
            jax.ShapeDtypeStruct((_B, _D), jnp.float32),
            jax.ShapeDtypeStruct((_B, _D), jnp.float32),
            jax.ShapeDtypeStruct((_B * _NNEG, _D), jnp.float32),
        ),
        scratch_types=[
            pltpu.VMEM((2 * _PC_PER_W + _NEG_PER_W,), jnp.int32),
            pltpu.VMEM((_CHUNK, _D), jnp.float32),
            pltpu.VMEM((_CHUNK, _D), jnp.float32),
            pltpu.SemaphoreType.DMA,
            pltpu.SemaphoreType.DMA,
        ],
    )(center_W, context_W, pc_idx, pctx_idx, neg_idx)


def kernel(center_W, context_W, pos_center, pos_context, neg_context):
    pc_idx = pos_center.astype(jnp.int32)
    pctx_idx = pos_context.astype(jnp.int32)
    neg_idx = neg_context.reshape(-1).astype(jnp.int32)
    pc, pctx, nctx = _fasttext_gather(center_W, context_W, pc_idx, pctx_idx, neg_idx)
    return pc, pctx, nctx.reshape(_B, _NNEG, _D)


# speedup vs baseline: 1.0090x; 1.0090x over previous
import functools

import jax
import jax.numpy as jnp
from jax import lax
from jax.experimental import pallas as pl
from jax.experimental.pallas import tpu as pltpu
from jax.experimental.pallas import tpu_sc as plsc

_B = 16384
_D = 64
_NNEG = 5

_info = plsc.get_sparse_core_info()
_NC = _info.num_cores
_NS = _info.num_subcores
_NW = _NC * _NS

_PC_PER_W = _B // _NW
_NEG_PER_W = _B * _NNEG // _NW
_CHUNK = 512


def _wid():
    return lax.axis_index("s") * _NC + lax.axis_index("c")


def _center_body(table_hbm, idx_hbm, out_hbm, idx_v, rows_v, sem):
    base = _wid() * _PC_PER_W
    pltpu.sync_copy(idx_hbm.at[pl.ds(base, _PC_PER_W)], idx_v)
    pltpu.async_copy(table_hbm.at[idx_v], rows_v, sem).wait()
    pltpu.sync_copy(rows_v, out_hbm.at[pl.ds(base, _PC_PER_W)])


def _context_body(table_hbm, pctx_idx_hbm, neg_idx_hbm, pctx_out, neg_out,
                  idx_v, rows_a, rows_b, sem_a, sem_b):
    w = _wid()
    base = w * _PC_PER_W
    nbase = w * _NEG_PER_W

    pltpu.sync_copy(pctx_idx_hbm.at[pl.ds(base, _PC_PER_W)],
                    idx_v.at[pl.ds(0, _PC_PER_W)])
    pltpu.sync_copy(neg_idx_hbm.at[pl.ds(nbase, _NEG_PER_W)],
                    idx_v.at[pl.ds(_PC_PER_W, _NEG_PER_W)])

    plan = [(0, pctx_out, base)]
    for j in range(_NNEG):
        plan.append(((1 + j) * _CHUNK, neg_out, nbase + j * _CHUNK))

    bufs = (rows_a, rows_b)
    sems = (sem_a, sem_b)

    def gather(c):
        ioff = plan[c][0]
        return pltpu.async_copy(
            table_hbm.at[idx_v.at[pl.ds(ioff, _CHUNK)]], bufs[c % 2], sems[c % 2])

    def store(c):
        _, out, ooff = plan[c]
        pltpu.sync_copy(bufs[c % 2], out.at[pl.ds(ooff, _CHUNK)])

    h = gather(0)
    for c in range(1, len(plan)):
        h_next = gather(c)
        h.wait()
        store(c - 1)
        h = h_next
    h.wait()
    store(len(plan) - 1)


_SC_PARAMS = pltpu.CompilerParams(use_tc_tiling_on_sc=False)


@jax.jit
def _fasttext_gather(center_W, context_W, pc_idx, pctx_idx, neg_idx):
    mesh = plsc.VectorSubcoreMesh(core_axis_name="c", subcore_axis_name="s")
    pc = pl.kernel(
        _center_body,
        mesh=mesh,
        compiler_params=_SC_PARAMS,
        out_type=jax.ShapeDtypeStruct((_B, _D), jnp.float32),
        scratch_types=[
            pltpu.VMEM((_PC_PER_W,), jnp.int32),
            pltpu.VMEM((_PC_PER_W, _D), jnp.float32),
            pltpu.SemaphoreType.DMA,
        ],
    )(center_W, pc_idx)
    pctx, nctx = pl.kernel(
        _context_body,
        mesh=mesh,
        compiler_params=_SC_PARAMS,
        out_type=(
            jax.ShapeDtypeStruct((_B, _D), jnp.float32),
            jax.ShapeDtypeStruct((_B * _NNEG, _D), jnp.float32),
        ),
        scratch_types=[
            pltpu.VMEM((_PC_PER_W + _NEG_PER_W,), jnp.int32),
            pltpu.VMEM((_CHUNK, _D), jnp.float32),
            pltpu.VMEM((_CHUNK, _D), jnp.float32),
            pltpu.SemaphoreType.DMA,
            pltpu.SemaphoreType.DMA,
        ],
    )(context_W, pctx_idx, neg_idx)
    return pc, pctx, nctx


def kernel(center_W, context_W, pos_center, pos_context, neg_context):
    pc_idx = pos_center.astype(jnp.int32)
    pctx_idx = pos_context.astype(jnp.int32)
    neg_idx = neg_context.reshape(-1).astype(jnp.int32)
    pc, pctx, nctx = _fasttext_gather(center_W, context_W, pc_idx, pctx_idx, neg_idx)
    return pc, pctx, nctx.reshape(_B, _NNEG, _D)


# speedup vs baseline: 2.0215x; 2.0036x over previous
import functools

import jax
import jax.numpy as jnp
from jax import lax
from jax.experimental import pallas as pl
from jax.experimental.pallas import tpu as pltpu
from jax.experimental.pallas import tpu_sc as plsc

_B = 16384
_D = 64
_NNEG = 5
_V = 1000000

_info = plsc.get_sparse_core_info()
_NC = _info.num_cores
_NS = _info.num_subcores
_NW = _NC * _NS

_PC_PER_W = _B // _NW
_NEG_PER_W = _B * _NNEG // _NW
_CHUNK = 256


def _section(table3, idx_v, idx_off, out_hbm, out_base, n_chunks,
             rows_v, sem):
    def chunk(c, _):
        coff = idx_off + c * _CHUNK
        copies = []
        for k in range(_CHUNK):
            if k % 16 == 0:
                iv = idx_v[pl.ds(coff + k, 16)]
            i = iv[k % 16]
            copies.append(pltpu.async_copy(
                table3.at[i >> 3, i & 7], rows_v.at[k], sem))
        for h in copies:
            h.wait()
        pltpu.sync_copy(rows_v, out_hbm.at[pl.ds(out_base + c * _CHUNK, _CHUNK)])
        return _

    lax.fori_loop(0, n_chunks, chunk, 0)


def _sc_body(center3, context3, pc_idx_hbm, pctx_idx_hbm, neg_idx_hbm,
             pc_out, pctx_out, neg_out,
             idx_v, rows_v, sem):
    w = lax.axis_index("s") * _NC + lax.axis_index("c")
    base = w * _PC_PER_W
    nbase = w * _NEG_PER_W

    pltpu.sync_copy(pc_idx_hbm.at[pl.ds(base, _PC_PER_W)],
                    idx_v.at[pl.ds(0, _PC_PER_W)])
    pltpu.sync_copy(pctx_idx_hbm.at[pl.ds(base, _PC_PER_W)],
                    idx_v.at[pl.ds(_PC_PER_W, _PC_PER_W)])
    pltpu.sync_copy(neg_idx_hbm.at[pl.ds(nbase, _NEG_PER_W)],
                    idx_v.at[pl.ds(2 * _PC_PER_W, _NEG_PER_W)])

    _section(center3, idx_v, 0, pc_out, base, _PC_PER_W // _CHUNK, rows_v, sem)
    _section(context3, idx_v, _PC_PER_W, pctx_out, base,
             _PC_PER_W // _CHUNK, rows_v, sem)
    _section(context3, idx_v, 2 * _PC_PER_W, neg_out, nbase,
             _NEG_PER_W // _CHUNK, rows_v, sem)


@jax.jit
def _fasttext_gather(center_W, context_W, pc_idx, pctx_idx, neg_idx):
    center3 = center_W.reshape(_V // 8, 8, _D)
    context3 = context_W.reshape(_V // 8, 8, _D)
    mesh = plsc.VectorSubcoreMesh(core_axis_name="c", subcore_axis_name="s")
    return pl.kernel(
        _sc_body,
        mesh=mesh,
        out_type=(
            jax.ShapeDtypeStruct((_B, _D), jnp.float32),
            jax.ShapeDtypeStruct((_B, _D), jnp.float32),
            jax.ShapeDtypeStruct((_B * _NNEG, _D), jnp.float32),
        ),
        scratch_types=[
            pltpu.VMEM((2 * _PC_PER_W + _NEG_PER_W,), jnp.int32),
            pltpu.VMEM((_CHUNK, _D), jnp.float32),
            pltpu.SemaphoreType.DMA,
        ],
    )(center3, context3, pc_idx, pctx_idx, neg_idx)


def kernel(center_W, context_W, pos_center, pos_context, neg_context):
    pc_idx = pos_center.astype(jnp.int32)
    pctx_idx = pos_context.astype(jnp.int32)
    neg_idx = neg_context.reshape(-1).astype(jnp.int32)
    pc, pctx, nctx = _fasttext_gather(center_W, context_W, pc_idx, pctx_idx, neg_idx)
    return pc, pctx, nctx.reshape(_B, _NNEG, _D)
